# baseline (speedup 1.0000x reference)
import functools

import jax
import jax.numpy as jnp
from jax import lax
from jax.experimental import pallas as pl
from jax.experimental.pallas import tpu as pltpu
from jax.experimental.pallas import tpu_sc as plsc

B, C, H, W_DIM = 16, 96, 224, 224
HW = H * W_DIM
M = int(HW * 0.25)
NCHUNK = 8
S = HW // NCHUNK
NVEC = HW // 16

CPW = C // 2
SEG = 512
NSEG = HW // SEG
VSEG = SEG // 16

_f32 = jnp.float32
_i32 = jnp.int32


def _att_body(fm_ref, w_ref, b_ref, att_ref):
    x = fm_ref[0]
    w = w_ref[...]
    acc = lax.dot_general(w, x, (((1,), (0,)), ((), ())),
                          preferred_element_type=_f32)
    att_ref[0] = jax.nn.sigmoid(acc + b_ref[0, 0])


def _compute_att(fm3, w2, b2):
    return pl.pallas_call(
        _att_body,
        grid=(B,),
        in_specs=[
            pl.BlockSpec((1, C, HW), lambda i: (i, 0, 0)),
            pl.BlockSpec((1, C), lambda i: (0, 0)),
            pl.BlockSpec(memory_space=pltpu.SMEM),
        ],
        out_specs=pl.BlockSpec((1, 1, HW), lambda i: (i, 0, 0)),
        out_shape=jax.ShapeDtypeStruct((B, 1, HW), _f32),
    )(fm3, w2, b2)


def _radix_select(att_v, hist_v):
    lanes = lax.iota(_i32, 16)
    ones = jnp.ones((16,), _f32)
    zeros = jnp.zeros((16,), _f32)
    m255 = lax.broadcast(jnp.asarray(255, _i32), (16,))

    def histogram(shift, prefix, check):
        def zero_body(i, _):
            hist_v[pl.ds(i * 16, 16)] = zeros
            return 0
        lax.fori_loop(0, 16, zero_body, 0)
        sh_v = lax.broadcast(jnp.asarray(shift, _i32), (16,))
        shp_v = lax.broadcast(jnp.asarray(shift + 8, _i32), (16,))
        pv = lax.broadcast(prefix, (16,))

        def body(j, _):
            v = att_v[pl.ds(j * 16, 16)]
            bits = lax.bitcast_convert_type(v, _i32)
            bn = lax.shift_right_logical(bits, sh_v) & m255
            if check:
                msk = lax.shift_right_logical(bits, shp_v) == pv
                plsc.addupdate_scatter(hist_v, [bn], ones, mask=msk)
            else:
                plsc.addupdate_scatter(hist_v, [bn], ones)
            return 0
        lax.fori_loop(0, NVEC, body, 0)

    def pick_bin(rem):
        def bs(i, acc):
            hv = hist_v[pl.ds(i * 16, 16)]
            s = jnp.sum(hv)
            iv = lax.broadcast(i, (16,))
            return acc + jnp.where(lanes == iv, lax.broadcast(s, (16,)), zeros)
        bsum = lax.fori_loop(0, 16, bs, zeros)
        gsb = jnp.flip(jnp.cumsum(jnp.flip(bsum)))
        remv = lax.broadcast(rem, (16,))
        istar = (jnp.sum(jnp.where(gsb >= remv, ones, zeros)) - 1.0).astype(_i32)
        iv = lax.broadcast(istar, (16,))
        after = (jnp.sum(jnp.where(lanes == iv, gsb, zeros))
                 - jnp.sum(jnp.where(lanes == iv, bsum, zeros)))
        selv = hist_v[pl.ds(istar * 16, 16)]
        wgs = jnp.flip(jnp.cumsum(jnp.flip(selv))) + lax.broadcast(after, (16,))
        jstar = (jnp.sum(jnp.where(wgs >= remv, ones, zeros)) - 1.0).astype(_i32)
        jv = lax.broadcast(jstar, (16,))
        sel_wgs = jnp.sum(jnp.where(lanes == jv, wgs, zeros))
        sel_h = jnp.sum(jnp.where(lanes == jv, selv, zeros))
        return istar * 16 + jstar, rem - (sel_wgs - sel_h)

    rem = jnp.asarray(float(M), _f32)
    prefix = jnp.asarray(0, _i32)
    for lvl, shift in enumerate((24, 16, 8, 0)):
        histogram(shift, prefix, check=(lvl > 0))
        binstar, rem = pick_bin(rem)
        prefix = prefix * 256 + binstar
    thr_vec = lax.bitcast_convert_type(lax.broadcast(prefix, (16,)), _f32)
    return thr_vec


def _sc_body(fm_hbm, att_hbm, out_hbm, att_v, hist_v, buf0, buf1,
             si0, si1, so0, so1):
    w = lax.axis_index("s") * 2 + lax.axis_index("c")
    b = w // 2
    ch0 = (w % 2) * CPW

    def chunk_src(k):
        return fm_hbm.at[b, pl.ds(ch0, CPW), pl.ds(k * SEG, SEG)]

    def chunk_dst(k):
        return out_hbm.at[b, pl.ds(ch0, CPW), pl.ds(k * SEG, SEG)]

    in0 = pltpu.make_async_copy(chunk_src(0), buf0, si0)
    in0.start()
    in1 = pltpu.make_async_copy(chunk_src(1), buf1, si1)
    in1.start()

    pltpu.sync_copy(att_hbm.at[b, 0], att_v)
    thrv = _radix_select(att_v, hist_v)
    fzeros = jnp.zeros((16,), _f32)

    def compute(buf, k):
        qbase = k * SEG

        def inner(v, _):
            off = v * 16
            a = att_v[pl.ds(qbase + off, 16)]
            keep = a < thrv
            for c in range(CPW):
                x = buf[c, pl.ds(off, 16)]
                buf[c, pl.ds(off, 16)] = jnp.where(keep, x, fzeros)
            return 0
        lax.fori_loop(0, VSEG, inner, 0)

    def pair(kk, _):
        k0 = kk * 2
        k1 = k0 + 1
        pltpu.make_async_copy(chunk_src(k0), buf0, si0).wait()
        compute(buf0, k0)
        pltpu.make_async_copy(buf0, chunk_dst(k0), so0).start()
        pltpu.make_async_copy(chunk_src(k1), buf1, si1).wait()
        compute(buf1, k1)
        pltpu.make_async_copy(buf1, chunk_dst(k1), so1).start()

        @pl.when(kk < NSEG // 2 - 1)
        def _():
            pltpu.make_async_copy(buf0, chunk_dst(k0), so0).wait()
            pltpu.make_async_copy(chunk_src(k0 + 2), buf0, si0).start()
            pltpu.make_async_copy(buf1, chunk_dst(k1), so1).wait()
            pltpu.make_async_copy(chunk_src(k1 + 2), buf1, si1).start()
        return 0

    lax.fori_loop(0, NSEG // 2, pair, 0)
    pltpu.make_async_copy(buf0, chunk_dst(NSEG - 2), so0).wait()
    pltpu.make_async_copy(buf1, chunk_dst(NSEG - 1), so1).wait()


_sc_drop = functools.partial(
    pl.kernel,
    out_type=jax.ShapeDtypeStruct((B, C, HW), _f32),
    mesh=plsc.VectorSubcoreMesh(core_axis_name="c", subcore_axis_name="s",
                                num_cores=2, num_subcores=16),
    compiler_params=pltpu.CompilerParams(needs_layout_passes=False),
    scratch_types=[
        pltpu.VMEM((HW,), _f32),
        pltpu.VMEM((256,), _f32),
        pltpu.VMEM((CPW, SEG), _f32),
        pltpu.VMEM((CPW, SEG), _f32),
        pltpu.SemaphoreType.DMA,
        pltpu.SemaphoreType.DMA,
        pltpu.SemaphoreType.DMA,
        pltpu.SemaphoreType.DMA,
    ],
)(_sc_body)


def kernel(feature_maps, W, b):
    fm3 = feature_maps.reshape(B, C, HW)
    w2 = W.reshape(1, C)
    b2 = b.reshape(1, 1)
    att3 = _compute_att(fm3, w2, b2)
    out = _sc_drop(fm3, att3)
    return (out.reshape(B, C, H, W_DIM),
            att3.reshape(B, 1, H, W_DIM))


# speedup vs baseline: 2.3810x; 2.3810x over previous
import functools

import jax
import jax.numpy as jnp
from jax import lax
from jax.experimental import pallas as pl
from jax.experimental.pallas import tpu as pltpu
from jax.experimental.pallas import tpu_sc as plsc

B, C, H, W_DIM = 16, 96, 224, 224
HW = H * W_DIM
M = int(HW * 0.25)
NVEC = HW // 16
HSPL = 2
HH = H // HSPL

_f32 = jnp.float32
_i32 = jnp.int32
_bf16 = jnp.bfloat16


def _att_body(fm_ref, w_ref, b_ref, att_ref):
    x = fm_ref[0]
    acc = jnp.zeros((H, W_DIM), _f32)
    for c in range(C):
        xb = x[c].astype(_bf16).astype(_f32)
        wb = jnp.asarray(w_ref[0, c], _f32)
        wb = wb.astype(_bf16).astype(_f32)
        acc = acc + xb * wb
    att_ref[0, 0] = jax.nn.sigmoid(acc + b_ref[0, 0])


def _compute_att(fm, w2, b2):
    return pl.pallas_call(
        _att_body,
        grid=(B,),
        in_specs=[
            pl.BlockSpec((1, C, H, W_DIM), lambda i: (i, 0, 0, 0)),
            pl.BlockSpec(memory_space=pltpu.SMEM),
            pl.BlockSpec(memory_space=pltpu.SMEM),
        ],
        out_specs=pl.BlockSpec((1, 1, H, W_DIM), lambda i: (i, 0, 0, 0)),
        out_shape=jax.ShapeDtypeStruct((B, 1, H, W_DIM), _f32),
    )(fm, w2, b2)


def _sc_body(att_hbm, out_hbm, data_v, hist_v, tvec_v):
    wid = lax.axis_index("s") * 2 + lax.axis_index("c")

    @pl.when(wid < B)
    def _():
        pltpu.sync_copy(att_hbm.at[wid], data_v)

        lanes = lax.iota(_i32, 16)
        ones = jnp.ones((16,), _f32)
        zeros = jnp.zeros((16,), _f32)
        m255 = lax.broadcast(jnp.asarray(255, _i32), (16,))

        def histogram(shift, prefix, check):
            def zero_body(i, _):
                hist_v[pl.ds(i * 16, 16)] = zeros
                return 0
            lax.fori_loop(0, 16, zero_body, 0)
            sh_v = lax.broadcast(jnp.asarray(shift, _i32), (16,))
            shp_v = lax.broadcast(jnp.asarray(shift + 8, _i32), (16,))
            pv = lax.broadcast(prefix, (16,))

            def body(j, _):
                v = data_v[pl.ds(j * 16, 16)]
                bits = lax.bitcast_convert_type(v, _i32)
                bn = lax.shift_right_logical(bits, sh_v) & m255
                if check:
                    msk = lax.shift_right_logical(bits, shp_v) == pv
                    plsc.addupdate_scatter(hist_v, [bn], ones, mask=msk)
                else:
                    plsc.addupdate_scatter(hist_v, [bn], ones)
                return 0
            lax.fori_loop(0, NVEC, body, 0)

        def pick_bin(rem):
            def bs(i, acc):
                hv = hist_v[pl.ds(i * 16, 16)]
                s = jnp.sum(hv)
                iv = lax.broadcast(i, (16,))
                return acc + jnp.where(lanes == iv, lax.broadcast(s, (16,)), zeros)
            bsum = lax.fori_loop(0, 16, bs, zeros)
            gsb = jnp.flip(jnp.cumsum(jnp.flip(bsum)))
            remv = lax.broadcast(rem, (16,))
            istar = (jnp.sum(jnp.where(gsb >= remv, ones, zeros)) - 1.0).astype(_i32)
            iv = lax.broadcast(istar, (16,))
            after = (jnp.sum(jnp.where(lanes == iv, gsb, zeros))
                     - jnp.sum(jnp.where(lanes == iv, bsum, zeros)))
            selv = hist_v[pl.ds(istar * 16, 16)]
            wgs = jnp.flip(jnp.cumsum(jnp.flip(selv))) + lax.broadcast(after, (16,))
            jstar = (jnp.sum(jnp.where(wgs >= remv, ones, zeros)) - 1.0).astype(_i32)
            jv = lax.broadcast(jstar, (16,))
            sel_wgs = jnp.sum(jnp.where(lanes == jv, wgs, zeros))
            sel_h = jnp.sum(jnp.where(lanes == jv, selv, zeros))
            return istar * 16 + jstar, rem - (sel_wgs - sel_h)

        rem = jnp.asarray(float(M), _f32)
        prefix = jnp.asarray(0, _i32)
        for lvl, shift in enumerate((24, 16, 8, 0)):
            histogram(shift, prefix, check=(lvl > 0))
            binstar, rem = pick_bin(rem)
            prefix = prefix * 256 + binstar

        tvec_v[...] = lax.bitcast_convert_type(lax.broadcast(prefix, (16,)), _f32)
        pltpu.sync_copy(tvec_v, out_hbm.at[wid])


_sc_select = functools.partial(
    pl.kernel,
    out_type=jax.ShapeDtypeStruct((B, 16), _f32),
    mesh=plsc.VectorSubcoreMesh(core_axis_name="c", subcore_axis_name="s",
                                num_cores=2, num_subcores=16),
    compiler_params=pltpu.CompilerParams(needs_layout_passes=False),
    scratch_types=[
        pltpu.VMEM((HW,), _f32),
        pltpu.VMEM((256,), _f32),
        pltpu.VMEM((16,), _f32),
    ],
)(_sc_body)


def _mask_body(fm_ref, att_ref, thr_ref, out_ref):
    t = thr_ref[pl.program_id(0), 0]
    keep = (att_ref[0] < t).astype(_f32)
    out_ref[0] = fm_ref[0] * keep


def _apply_mask(fm, att4, thr):
    return pl.pallas_call(
        _mask_body,
        grid=(B, HSPL),
        in_specs=[
            pl.BlockSpec((1, C, HH, W_DIM), lambda i, j: (i, 0, j, 0)),
            pl.BlockSpec((1, 1, HH, W_DIM), lambda i, j: (i, 0, j, 0)),
            pl.BlockSpec(memory_space=pltpu.SMEM),
        ],
        out_specs=pl.BlockSpec((1, C, HH, W_DIM), lambda i, j: (i, 0, j, 0)),
        out_shape=jax.ShapeDtypeStruct((B, C, H, W_DIM), _f32),
    )(fm, att4, thr)


def kernel(feature_maps, W, b):
    w2 = W.reshape(1, C)
    b2 = b.reshape(1, 1)
    att4 = _compute_att(feature_maps, w2, b2)
    thr = _sc_select(att4.reshape(B, HW))
    out = _apply_mask(feature_maps, att4, thr)
    return (out, att4)


# speedup vs baseline: 2.4171x; 1.0151x over previous
import functools

import jax
import jax.numpy as jnp
from jax import lax
from jax.experimental import pallas as pl
from jax.experimental.pallas import tpu as pltpu
from jax.experimental.pallas import tpu_sc as plsc

B, C, H, W_DIM = 16, 96, 224, 224
HW = H * W_DIM
M = int(HW * 0.25)
NVEC = HW // 16
HSPL = 2
HH = H // HSPL

_f32 = jnp.float32
_i32 = jnp.int32
_bf16 = jnp.bfloat16


def _att_body(fm_ref, w_ref, b_ref, att_ref):
    x = fm_ref[0]
    acc = jnp.zeros((H, W_DIM), _f32)
    for c in range(C):
        xb = x[c].astype(_bf16).astype(_f32)
        wb = jnp.asarray(w_ref[0, c], _f32)
        wb = wb.astype(_bf16).astype(_f32)
        acc = acc + xb * wb
    att_ref[0, 0] = jax.nn.sigmoid(acc + b_ref[0, 0])


def _compute_att(fm, w2, b2):
    return pl.pallas_call(
        _att_body,
        grid=(B,),
        in_specs=[
            pl.BlockSpec((1, C, H, W_DIM), lambda i: (i, 0, 0, 0)),
            pl.BlockSpec(memory_space=pltpu.SMEM),
            pl.BlockSpec(memory_space=pltpu.SMEM),
        ],
        out_specs=pl.BlockSpec((1, 1, H, W_DIM), lambda i: (i, 0, 0, 0)),
        out_shape=jax.ShapeDtypeStruct((B, 1, H, W_DIM), _f32),
    )(fm, w2, b2)


def _sc_body(att_hbm, out_hbm, data_v, hist_v, tvec_v):
    wid = lax.axis_index("s") * 2 + lax.axis_index("c")

    @pl.when(wid < B)
    def _():
        pltpu.sync_copy(att_hbm.at[wid], data_v)

        lanes = lax.iota(_i32, 16)
        ones = jnp.ones((16,), _f32)
        zeros = jnp.zeros((16,), _f32)
        m255 = lax.broadcast(jnp.asarray(255, _i32), (16,))

        def histogram(shift, prefix, check):
            def zero_body(i, _):
                hist_v[pl.ds(i * 16, 16)] = zeros
                return 0
            lax.fori_loop(0, 16, zero_body, 0)
            sh_v = lax.broadcast(jnp.asarray(shift, _i32), (16,))
            shp_v = lax.broadcast(jnp.asarray(shift + 8, _i32), (16,))
            pv = lax.broadcast(prefix, (16,))

            def body(j, _):
                for u in range(8):
                    v = data_v[pl.ds((j * 8 + u) * 16, 16)]
                    bits = lax.bitcast_convert_type(v, _i32)
                    bn = lax.shift_right_logical(bits, sh_v) & m255
                    if check:
                        msk = lax.shift_right_logical(bits, shp_v) == pv
                        plsc.addupdate_scatter(hist_v, [bn], ones, mask=msk)
                    else:
                        plsc.addupdate_scatter(hist_v, [bn], ones)
                return 0
            lax.fori_loop(0, NVEC // 8, body, 0)

        def pick_bin(rem):
            def bs(i, acc):
                hv = hist_v[pl.ds(i * 16, 16)]
                s = jnp.sum(hv)
                iv = lax.broadcast(i, (16,))
                return acc + jnp.where(lanes == iv, lax.broadcast(s, (16,)), zeros)
            bsum = lax.fori_loop(0, 16, bs, zeros)
            gsb = jnp.flip(jnp.cumsum(jnp.flip(bsum)))
            remv = lax.broadcast(rem, (16,))
            istar = (jnp.sum(jnp.where(gsb >= remv, ones, zeros)) - 1.0).astype(_i32)
            iv = lax.broadcast(istar, (16,))
            after = (jnp.sum(jnp.where(lanes == iv, gsb, zeros))
                     - jnp.sum(jnp.where(lanes == iv, bsum, zeros)))
            selv = hist_v[pl.ds(istar * 16, 16)]
            wgs = jnp.flip(jnp.cumsum(jnp.flip(selv))) + lax.broadcast(after, (16,))
            jstar = (jnp.sum(jnp.where(wgs >= remv, ones, zeros)) - 1.0).astype(_i32)
            jv = lax.broadcast(jstar, (16,))
            sel_wgs = jnp.sum(jnp.where(lanes == jv, wgs, zeros))
            sel_h = jnp.sum(jnp.where(lanes == jv, selv, zeros))
            return istar * 16 + jstar, rem - (sel_wgs - sel_h)

        rem = jnp.asarray(float(M), _f32)
        prefix = jnp.asarray(0, _i32)
        for lvl, shift in enumerate((24, 16, 8, 0)):
            histogram(shift, prefix, check=(lvl > 0))
            binstar, rem = pick_bin(rem)
            prefix = prefix * 256 + binstar

        tvec_v[...] = lax.bitcast_convert_type(lax.broadcast(prefix, (16,)), _f32)
        pltpu.sync_copy(tvec_v, out_hbm.at[wid])


_sc_select = functools.partial(
    pl.kernel,
    out_type=jax.ShapeDtypeStruct((B, 16), _f32),
    mesh=plsc.VectorSubcoreMesh(core_axis_name="c", subcore_axis_name="s",
                                num_cores=2, num_subcores=16),
    compiler_params=pltpu.CompilerParams(needs_layout_passes=False),
    scratch_types=[
        pltpu.VMEM((HW,), _f32),
        pltpu.VMEM((256,), _f32),
        pltpu.VMEM((16,), _f32),
    ],
)(_sc_body)


def _mask_body(fm_ref, att_ref, thr_ref, out_ref):
    t = thr_ref[pl.program_id(0), 0]
    keep = (att_ref[0] < t).astype(_f32)
    out_ref[0] = fm_ref[0] * keep


def _apply_mask(fm, att4, thr):
    return pl.pallas_call(
        _mask_body,
        grid=(B, HSPL),
        in_specs=[
            pl.BlockSpec((1, C, HH, W_DIM), lambda i, j: (i, 0, j, 0)),
            pl.BlockSpec((1, 1, HH, W_DIM), lambda i, j: (i, 0, j, 0)),
            pl.BlockSpec(memory_space=pltpu.SMEM),
        ],
        out_specs=pl.BlockSpec((1, C, HH, W_DIM), lambda i, j: (i, 0, j, 0)),
        out_shape=jax.ShapeDtypeStruct((B, C, H, W_DIM), _f32),
    )(fm, att4, thr)


def kernel(feature_maps, W, b):
    w2 = W.reshape(1, C)
    b2 = b.reshape(1, 1)
    att4 = _compute_att(feature_maps, w2, b2)
    thr = _sc_select(att4.reshape(B, HW))
    out = _apply_mask(feature_maps, att4, thr)
    return (out, att4)


# speedup vs baseline: 2.7565x; 1.1404x over previous
import functools

import jax
import jax.numpy as jnp
from jax import lax
from jax.experimental import pallas as pl
from jax.experimental.pallas import tpu as pltpu
from jax.experimental.pallas import tpu_sc as plsc

B, C, H, W_DIM = 16, 96, 224, 224
HW = H * W_DIM
M = int(HW * 0.25)
NVEC = HW // 16
HSPL = 2
HH = H // HSPL

_f32 = jnp.float32
_i32 = jnp.int32
_bf16 = jnp.bfloat16


def _att_body(fm_ref, w_ref, b_ref, att_ref):
    x = fm_ref[0]
    acc = jnp.zeros((H, W_DIM), _f32)
    for c in range(C):
        xb = x[c].astype(_bf16).astype(_f32)
        wb = jnp.asarray(w_ref[0, c], _f32)
        wb = wb.astype(_bf16).astype(_f32)
        acc = acc + xb * wb
    att_ref[0, 0] = jax.nn.sigmoid(acc + b_ref[0, 0])


def _compute_att(fm, w2, b2):
    return pl.pallas_call(
        _att_body,
        grid=(B,),
        in_specs=[
            pl.BlockSpec((1, C, H, W_DIM), lambda i: (i, 0, 0, 0)),
            pl.BlockSpec(memory_space=pltpu.SMEM),
            pl.BlockSpec(memory_space=pltpu.SMEM),
        ],
        out_specs=pl.BlockSpec((1, 1, H, W_DIM), lambda i: (i, 0, 0, 0)),
        out_shape=jax.ShapeDtypeStruct((B, 1, H, W_DIM), _f32),
    )(fm, w2, b2)


HALF = HW // 2
NVEC2 = HALF // 16


def _sc_body(att_hbm, out_hbm, data_v, hist_v, tmp_v, tvec_v, shared_v):
    cc = lax.axis_index("c")
    ss = lax.axis_index("s")
    b = cc * 8 + ss // 2
    hf = ss % 2

    pltpu.sync_copy(att_hbm.at[b, pl.ds(hf * HALF, HALF)], data_v)

    lanes = lax.iota(_i32, 16)
    ones = jnp.ones((16,), _f32)
    zeros = jnp.zeros((16,), _f32)
    m255 = lax.broadcast(jnp.asarray(255, _i32), (16,))

    if True:
        def histogram(shift, prefix, check):
            def zero_body(i, _):
                hist_v[pl.ds(i * 16, 16)] = zeros
                return 0
            lax.fori_loop(0, 16, zero_body, 0)
            sh_v = lax.broadcast(jnp.asarray(shift, _i32), (16,))
            shp_v = lax.broadcast(jnp.asarray(shift + 8, _i32), (16,))
            pv = lax.broadcast(prefix, (16,))

            def body(j, _):
                for u in range(8):
                    v = data_v[pl.ds((j * 8 + u) * 16, 16)]
                    bits = lax.bitcast_convert_type(v, _i32)
                    bn = lax.shift_right_logical(bits, sh_v) & m255
                    if check:
                        msk = lax.shift_right_logical(bits, shp_v) == pv
                        plsc.addupdate_scatter(hist_v, [bn], ones, mask=msk)
                    else:
                        plsc.addupdate_scatter(hist_v, [bn], ones)
                return 0
            lax.fori_loop(0, NVEC2 // 8, body, 0)
            plsc.subcore_barrier()
            pltpu.sync_copy(hist_v, shared_v.at[ss])
            plsc.subcore_barrier()
            pltpu.sync_copy(shared_v.at[ss ^ 1], tmp_v)
            for i in range(16):
                hist_v[pl.ds(i * 16, 16)] = (hist_v[pl.ds(i * 16, 16)]
                                             + tmp_v[pl.ds(i * 16, 16)])

        def pick_bin(rem):
            def bs(i, acc):
                hv = hist_v[pl.ds(i * 16, 16)]
                s = jnp.sum(hv)
                iv = lax.broadcast(i, (16,))
                return acc + jnp.where(lanes == iv, lax.broadcast(s, (16,)), zeros)
            bsum = lax.fori_loop(0, 16, bs, zeros)
            gsb = jnp.flip(jnp.cumsum(jnp.flip(bsum)))
            remv = lax.broadcast(rem, (16,))
            istar = (jnp.sum(jnp.where(gsb >= remv, ones, zeros)) - 1.0).astype(_i32)
            iv = lax.broadcast(istar, (16,))
            after = (jnp.sum(jnp.where(lanes == iv, gsb, zeros))
                     - jnp.sum(jnp.where(lanes == iv, bsum, zeros)))
            selv = hist_v[pl.ds(istar * 16, 16)]
            wgs = jnp.flip(jnp.cumsum(jnp.flip(selv))) + lax.broadcast(after, (16,))
            jstar = (jnp.sum(jnp.where(wgs >= remv, ones, zeros)) - 1.0).astype(_i32)
            jv = lax.broadcast(jstar, (16,))
            sel_wgs = jnp.sum(jnp.where(lanes == jv, wgs, zeros))
            sel_h = jnp.sum(jnp.where(lanes == jv, selv, zeros))
            return istar * 16 + jstar, rem - (sel_wgs - sel_h)

        rem = jnp.asarray(float(M), _f32)
        prefix = jnp.asarray(0, _i32)
        for lvl, shift in enumerate((24, 16, 8, 0)):
            histogram(shift, prefix, check=(lvl > 0))
            binstar, rem = pick_bin(rem)
            prefix = prefix * 256 + binstar

        tvec_v[...] = lax.bitcast_convert_type(lax.broadcast(prefix, (16,)), _f32)

        @pl.when(hf == 0)
        def _():
            pltpu.sync_copy(tvec_v, out_hbm.at[b])


_sc_select = functools.partial(
    pl.kernel,
    out_type=jax.ShapeDtypeStruct((B, 16), _f32),
    mesh=plsc.VectorSubcoreMesh(core_axis_name="c", subcore_axis_name="s",
                                num_cores=2, num_subcores=16),
    compiler_params=pltpu.CompilerParams(needs_layout_passes=False),
    scratch_types=[
        pltpu.VMEM((HALF,), _f32),
        pltpu.VMEM((256,), _f32),
        pltpu.VMEM((256,), _f32),
        pltpu.VMEM((16,), _f32),
        pltpu.VMEM_SHARED((16, 256), _f32),
    ],
)(_sc_body)


def _mask_body(fm_ref, att_ref, thr_ref, out_ref):
    t = thr_ref[pl.program_id(0), 0]
    keep = (att_ref[0] < t).astype(_f32)
    out_ref[0] = fm_ref[0] * keep


def _apply_mask(fm, att4, thr):
    return pl.pallas_call(
        _mask_body,
        grid=(B, HSPL),
        in_specs=[
            pl.BlockSpec((1, C, HH, W_DIM), lambda i, j: (i, 0, j, 0)),
            pl.BlockSpec((1, 1, HH, W_DIM), lambda i, j: (i, 0, j, 0)),
            pl.BlockSpec(memory_space=pltpu.SMEM),
        ],
        out_specs=pl.BlockSpec((1, C, HH, W_DIM), lambda i, j: (i, 0, j, 0)),
        out_shape=jax.ShapeDtypeStruct((B, C, H, W_DIM), _f32),
    )(fm, att4, thr)


def kernel(feature_maps, W, b):
    w2 = W.reshape(1, C)
    b2 = b.reshape(1, 1)
    att4 = _compute_att(feature_maps, w2, b2)
    thr = _sc_select(att4.reshape(B, HW))
    out = _apply_mask(feature_maps, att4, thr)
    return (out, att4)
